# baseline (speedup 1.0000x reference)
import jax
import jax.numpy as jnp
import numpy as np
from jax.experimental import pallas as pl
from jax.experimental.pallas import tpu as pltpu

W = 1024
H = 1024
S = 8
BH = 128

_INF = np.float32(np.inf)


def _rt_block(params_ref, out_ref):
    i = pl.program_id(0)
    row0 = (i * BH).astype(jnp.float32)

    rows = jax.lax.broadcasted_iota(jnp.int32, (BH, W), 0).astype(jnp.float32) + row0
    cols = jax.lax.broadcasted_iota(jnp.int32, (BH, W), 1).astype(jnp.float32)

    u = cols * np.float32(2.0 / (W - 1)) - np.float32(1.0)
    v = np.float32(1.0) - rows * np.float32(2.0 / (H - 1))
    inv_n = jax.lax.rsqrt(u * u + v * v + np.float32(25.0))

    best_t = bp = bir = bcl = bklr = bklg = bklb = None

    for s in range(S):
        o = 9 * s
        cx = params_ref[o + 0]
        cy5 = params_ref[o + 1]
        cz = params_ref[o + 2]
        cc = params_ref[o + 3]
        klr = params_ref[o + 4]
        klg = params_ref[o + 5]
        klb = params_ref[o + 6]
        ir = params_ref[o + 7]
        cl = params_ref[o + 8]

        q = u * cx + v * cz + cy5
        p = q * inv_n
        disc = p * p - cc
        t = p - disc * jax.lax.rsqrt(disc)

        if s == 0:
            best_t = jnp.where(disc >= np.float32(0.0), t, _INF)
            bp = p
            bir = jnp.full((BH, W), ir, dtype=jnp.float32)
            bcl = jnp.full((BH, W), cl, dtype=jnp.float32)
            bklr = jnp.full((BH, W), klr, dtype=jnp.float32)
            bklg = jnp.full((BH, W), klg, dtype=jnp.float32)
            bklb = jnp.full((BH, W), klb, dtype=jnp.float32)
        else:
            m = t < best_t
            best_t = jnp.where(m, t, best_t)
            bp = jnp.where(m, p, bp)
            bir = jnp.where(m, ir, bir)
            bcl = jnp.where(m, cl, bcl)
            bklr = jnp.where(m, klr, bklr)
            bklg = jnp.where(m, klg, bklg)
            bklb = jnp.where(m, klb, bklb)

    lx = params_ref[9 * S + 0]
    lz = params_ref[9 * S + 1]
    ly5 = params_ref[9 * S + 2]
    ll = params_ref[9 * S + 3]

    valid = best_t < _INF
    st = best_t

    qL = u * lx + v * lz + ly5
    pL = qL * inv_n

    st2 = st * st
    a_ = st * pL
    u1 = st * bp
    ams = a_ - st2
    dotnum = ams + (u1 - bcl)
    itl_arg = (ll - a_) - ams
    itl = jax.lax.rsqrt(jnp.maximum(itl_arg, np.float32(1e-20)))

    dot = dotnum * bir * itl
    diffuse = np.float32(0.5) * jnp.maximum(dot, np.float32(0.0))

    tsb = st - bp
    t2_ = dot * bir
    t3 = (t2_ + t2_) * tsb
    sd0 = itl * (pL - st) - t3
    sigma = jnp.copysign(np.float32(1.0), st)
    sd = jnp.maximum(sigma * sd0, np.float32(0.0))
    s2 = sd * sd
    s4 = s2 * s2
    s8 = s4 * s4
    spec = np.float32(0.8) * (s8 * s8)
    coef = np.float32(0.4) + diffuse + spec

    vb = rows * np.float32(2.0 / (H - 1)) - np.float32(1.0)
    one = np.float32(1.0)
    zero = np.float32(0.0)
    omv = one - vb
    bg_r = omv * np.float32(0.5) + vb
    bg_g = omv * np.float32(0.7) + vb
    bg_b = omv + vb

    col_r = jnp.clip(coef * bklr, zero, one)
    col_g = jnp.clip(coef * bklg, zero, one)
    col_b = jnp.clip(coef * bklb, zero, one)

    out_ref[0, :, :] = jnp.where(valid, col_r, bg_r)
    out_ref[1, :, :] = jnp.where(valid, col_g, bg_g)
    out_ref[2, :, :] = jnp.where(valid, col_b, bg_b)


def kernel(sphere_centers, sphere_radiuses, sphere_colors, light_pos, light_color):
    cc = jnp.sum(sphere_centers ** 2, axis=-1) - sphere_radiuses ** 2
    packed_centers = sphere_centers * jnp.array([1.0, 5.0, 1.0], jnp.float32)
    kl = sphere_colors * light_color[None, :]
    inv_r = 1.0 / jnp.maximum(sphere_radiuses, 1e-30)
    cl = sphere_centers @ light_pos
    per_sphere = jnp.concatenate(
        [packed_centers, cc[:, None], kl, inv_r[:, None], cl[:, None]], axis=-1
    ).reshape(-1)
    lscal = jnp.stack([
        light_pos[0],
        light_pos[2],
        5.0 * light_pos[1],
        jnp.sum(light_pos ** 2),
    ])
    params = jnp.concatenate([per_sphere, lscal]).astype(jnp.float32)

    img = pl.pallas_call(
        _rt_block,
        grid=(H // BH,),
        in_specs=[pl.BlockSpec(memory_space=pltpu.SMEM)],
        out_specs=pl.BlockSpec((3, BH, W), lambda i: (0, i, 0)),
        out_shape=jax.ShapeDtypeStruct((3, H, W), jnp.float32),
        compiler_params=pltpu.CompilerParams(
            dimension_semantics=("parallel",),
        ),
    )(params)
    return jnp.transpose(img, (1, 2, 0))


# speedup vs baseline: 1.0261x; 1.0261x over previous
import jax
import jax.numpy as jnp
import numpy as np
from jax.experimental import pallas as pl
from jax.experimental.pallas import tpu as pltpu

W = 1024
H = 1024
S = 8
BH = 128

_INF = np.float32(np.inf)


def _rt_block(params_ref, out_ref):
    i = pl.program_id(0)
    row0 = (i * BH).astype(jnp.float32)

    rows = jax.lax.broadcasted_iota(jnp.int32, (BH, W), 0).astype(jnp.float32) + row0
    cols = jax.lax.broadcasted_iota(jnp.int32, (BH, W), 1).astype(jnp.float32)

    u = cols * np.float32(2.0 / (W - 1)) - np.float32(1.0)
    v = np.float32(1.0) - rows * np.float32(2.0 / (H - 1))
    inv_n = jax.lax.rsqrt(u * u + v * v + np.float32(25.0))

    best_t = jnp.full((BH, W), _INF, dtype=jnp.float32)
    bcx = bcy = bcz = bklr = bklg = bklb = None

    for s in range(S):
        o = 8 * s
        cx = params_ref[o + 0]
        cy5 = params_ref[o + 1]
        cz = params_ref[o + 2]
        cc = params_ref[o + 3]
        klr = params_ref[o + 4]
        klg = params_ref[o + 5]
        klb = params_ref[o + 6]
        cy = params_ref[o + 7]

        q = u * cx + v * cz + cy5
        p = q * inv_n
        disc = p * p - cc
        t = p - disc * jax.lax.rsqrt(disc)

        if s == 0:
            best_t = jnp.where(disc >= np.float32(0.0), t, _INF)
            bcx = jnp.full((BH, W), cx, dtype=jnp.float32)
            bcy = jnp.full((BH, W), cy, dtype=jnp.float32)
            bcz = jnp.full((BH, W), cz, dtype=jnp.float32)
            bklr = jnp.full((BH, W), klr, dtype=jnp.float32)
            bklg = jnp.full((BH, W), klg, dtype=jnp.float32)
            bklb = jnp.full((BH, W), klb, dtype=jnp.float32)
        else:
            m = t < best_t
            best_t = jnp.where(m, t, best_t)
            bcx = jnp.where(m, cx, bcx)
            bcy = jnp.where(m, cy, bcy)
            bcz = jnp.where(m, cz, bcz)
            bklr = jnp.where(m, klr, bklr)
            bklg = jnp.where(m, klg, bklg)
            bklb = jnp.where(m, klb, bklb)

    lx = params_ref[8 * S + 0]
    ly = params_ref[8 * S + 1]
    lz = params_ref[8 * S + 2]

    valid = best_t < _INF
    sh = best_t * inv_n
    hx = sh * u
    hy = sh * np.float32(5.0)
    hz = sh * v

    nx = hx - bcx
    ny = hy - bcy
    nz = hz - bcz
    inn = jax.lax.rsqrt(nx * nx + ny * ny + nz * nz)
    nx = nx * inn
    ny = ny * inn
    nz = nz * inn

    tlx = lx - hx
    tly = ly - hy
    tlz = lz - hz
    itl = jax.lax.rsqrt(tlx * tlx + tly * tly + tlz * tlz)
    tlx = tlx * itl
    tly = tly * itl
    tlz = tlz * itl

    g = -jnp.copysign(inv_n, best_t)
    cmx = g * u
    cmy = g * np.float32(5.0)
    cmz = g * v

    dot = nx * tlx + ny * tly + nz * tlz
    diffuse = np.float32(0.5) * jnp.maximum(dot, np.float32(0.0))
    dot2 = dot + dot
    rx = nx * dot2 - tlx
    ry = ny * dot2 - tly
    rz = nz * dot2 - tlz
    sd = jnp.maximum(rx * cmx + ry * cmy + rz * cmz, np.float32(0.0))
    s2 = sd * sd
    s4 = s2 * s2
    s8 = s4 * s4
    spec = np.float32(0.8) * (s8 * s8)
    coef = np.float32(0.4) + diffuse + spec

    vb = rows * np.float32(2.0 / (H - 1)) - np.float32(1.0)
    one = np.float32(1.0)
    zero = np.float32(0.0)
    omv = one - vb
    bg_r = omv * np.float32(0.5) + vb
    bg_g = omv * np.float32(0.7) + vb
    bg_b = omv + vb

    col_r = jnp.clip(coef * bklr, zero, one)
    col_g = jnp.clip(coef * bklg, zero, one)
    col_b = jnp.clip(coef * bklb, zero, one)

    out_ref[0, :, :] = jnp.where(valid, col_r, bg_r)
    out_ref[1, :, :] = jnp.where(valid, col_g, bg_g)
    out_ref[2, :, :] = jnp.where(valid, col_b, bg_b)


def kernel(sphere_centers, sphere_radiuses, sphere_colors, light_pos, light_color):
    cc = jnp.sum(sphere_centers ** 2, axis=-1) - sphere_radiuses ** 2
    packed_centers = sphere_centers * jnp.array([1.0, 5.0, 1.0], jnp.float32)
    kl = sphere_colors * light_color[None, :]
    per_sphere = jnp.concatenate(
        [packed_centers, cc[:, None], kl, sphere_centers[:, 1:2]], axis=-1
    ).reshape(-1)
    params = jnp.concatenate([per_sphere, light_pos]).astype(jnp.float32)

    img = pl.pallas_call(
        _rt_block,
        grid=(H // BH,),
        in_specs=[pl.BlockSpec(memory_space=pltpu.SMEM)],
        out_specs=pl.BlockSpec((3, BH, W), lambda i: (0, i, 0)),
        out_shape=jax.ShapeDtypeStruct((3, H, W), jnp.float32),
        compiler_params=pltpu.CompilerParams(
            dimension_semantics=("parallel",),
        ),
    )(params)
    return jnp.transpose(img, (1, 2, 0))
